# baseline (speedup 1.0000x reference)
import jax
import jax.numpy as jnp
from jax.experimental import pallas as pl
from jax.experimental.pallas import tpu as pltpu
from jax.experimental.pallas import tpu_sc as plsc

_REG = 1e-5
_GATHER_WINDOW = 256
_TC_CHUNK = 2048


def _sc_gather(packed_table, idx):
    n = idx.shape[0]
    width = packed_table.shape[1]
    idx2 = idx.reshape(1, n)
    mesh = plsc.VectorSubcoreMesh(core_axis_name="core", subcore_axis_name="subcore")

    @pl.kernel(
        out_type=jax.ShapeDtypeStruct((n, width), packed_table.dtype),
        mesh=mesh,
    )
    def gather_kernel(x_hbm, i_hbm, o_hbm):
        def body(i_vmem, o_vmem):
            pltpu.sync_copy(x_hbm.at[i_vmem.at[0]], o_vmem)

        pltpu.emit_pipeline(
            body,
            grid=(n // _GATHER_WINDOW,),
            in_specs=[pl.BlockSpec((1, _GATHER_WINDOW), index_map=lambda i: (0, i))],
            out_specs=[pl.BlockSpec((_GATHER_WINDOW, width), index_map=lambda i: (i, 0))],
            core_axis_name=("core", "subcore"),
            dimension_semantics=(pltpu.PARALLEL,),
        )(i_hbm, o_hbm)

    return gather_kernel(packed_table, idx2)


def _tc_reduce(gathered_bits, quarter, batch):
    width = gathered_bits.shape[2]
    quarter_w = width // 4
    n_steps = gathered_bits.shape[1] // _TC_CHUNK

    def body(g_ref, q_ref, loss_ref, bpr_ref, emb_ref, acc_ref):
        i = pl.program_id(0)

        @pl.when(i == 0)
        def _():
            acc_ref[0] = 0.0
            acc_ref[1] = 0.0

        lane_q = jax.lax.broadcasted_iota(jnp.int32, (_TC_CHUNK, width), 1) // quarter_w

        def pick(k):
            q = q_ref[k][:, None]
            m = jnp.where(lane_q == q, g_ref[k], 0)
            y = m | pltpu.roll(m, 2 * quarter_w, 1)
            y = y | pltpu.roll(y, quarter_w, 1)
            y = y[:, :quarter_w]
            lo = jax.lax.bitcast_convert_type(y << 16, jnp.float32)
            hi = jax.lax.bitcast_convert_type(y & jnp.int32(-65536), jnp.float32)
            return lo, hi

        ulo, uhi = pick(0)
        plo, phi = pick(1)
        nlo, nhi = pick(2)
        d = jnp.sum(ulo * (plo - nlo) + uhi * (phi - nhi), axis=1)
        acc_ref[0] += jnp.sum(jax.nn.log_sigmoid(d.reshape(-1, 128)))
        acc_ref[1] += jnp.sum(ulo * ulo + uhi * uhi + plo * plo + phi * phi
                              + nlo * nlo + nhi * nhi)

        @pl.when(i == n_steps - 1)
        def _():
            bpr = -acc_ref[0] / batch
            emb = _REG * acc_ref[1] / (2.0 * batch)
            bpr_ref[0, 0] = bpr
            emb_ref[0, 0] = emb
            loss_ref[0, 0] = bpr + emb

    out_shape = [jax.ShapeDtypeStruct((1, 1), jnp.float32)] * 3
    smem = pl.BlockSpec(memory_space=pltpu.SMEM)
    loss, bpr, emb = pl.pallas_call(
        body,
        grid=(n_steps,),
        in_specs=[
            pl.BlockSpec((3, _TC_CHUNK, width), lambda i: (0, i, 0)),
            pl.BlockSpec((3, _TC_CHUNK), lambda i: (0, i)),
        ],
        out_shape=out_shape,
        out_specs=[smem, smem, smem],
        scratch_shapes=[pltpu.SMEM((2,), jnp.float32)],
    )(gathered_bits, quarter)
    return loss[0, 0], bpr[0, 0], emb[0, 0]


def kernel(all_embed, u_id, pos_i_id, neg_i_id):
    batch = u_id.shape[0]
    n_rows, emb = all_embed.shape
    a16 = all_embed.astype(jnp.bfloat16).reshape(n_rows, emb // 2, 2)
    packed = jax.lax.bitcast_convert_type(a16, jnp.float32).reshape(n_rows // 4, 2 * emb)
    idx = jnp.concatenate([u_id, pos_i_id, neg_i_id]).astype(jnp.int32)
    gathered = _sc_gather(packed, idx >> 2)
    gathered_bits = jax.lax.bitcast_convert_type(gathered, jnp.int32)
    gathered_bits = gathered_bits.reshape(3, batch, 2 * emb)
    quarter = (idx & 3).reshape(3, batch)
    loss, bpr, emb_loss = _tc_reduce(gathered_bits, quarter, float(batch))
    reward = jnp.float32(0.0)
    return (reward, loss, bpr, emb_loss)
